# Initial kernel scaffold; baseline (speedup 1.0000x reference)
import jax
import jax.numpy as jnp
from jax.experimental import pallas as pl


def kernel(group_id, table):
    raise NotImplementedError("write your pallas kernel here")


# speedup vs baseline: 1.0496x; 1.0496x over previous
import jax
import jax.numpy as jnp
from jax.experimental import pallas as pl
from jax.experimental.pallas import tpu as pltpu

_G = 256
_D = 64
_PACK = 2
_TR = 1024


def _onehot_kernel(ids_ref, tab_ref, out_ref):
    tr = ids_ref.shape[0]
    ids = ids_ref[...]
    col = jax.lax.broadcasted_iota(jnp.int32, (tr, _G), 1)
    oh0 = (col == ids[:, 0:1]).astype(tab_ref.dtype)
    oh1 = (col == ids[:, 1:2]).astype(tab_ref.dtype)
    oh = jnp.concatenate([oh0, oh1], axis=1)
    out_ref[...] = jnp.dot(oh, tab_ref[...],
                           preferred_element_type=jnp.float32)


def kernel(group_id, table):
    (B,) = group_id.shape
    rows = B // _PACK
    ids = group_id.astype(jnp.int32).reshape(rows, _PACK)
    num_tiles = rows // _TR

    out_packed = pl.pallas_call(
        _onehot_kernel,
        out_shape=jax.ShapeDtypeStruct((rows, _PACK * _D), jnp.float32),
        grid=(num_tiles,),
        in_specs=[
            pl.BlockSpec((_TR, _PACK), lambda i: (i, 0)),
            pl.BlockSpec((_PACK * _G, _PACK * _D), lambda i: (0, 0)),
        ],
        out_specs=pl.BlockSpec((_TR, _PACK * _D), lambda i: (i, 0)),
        compiler_params=pltpu.CompilerParams(
            dimension_semantics=("parallel",)),
    )(ids, table)

    return out_packed.reshape(B, _D)


# speedup vs baseline: 4.1026x; 3.9086x over previous
import jax
import jax.numpy as jnp
from jax.experimental import pallas as pl
from jax.experimental.pallas import tpu as pltpu

_G = 256
_D = 64
_TB = 2048


def _gather_t_kernel(ids_ref, tabt_ref, out_ref):
    pieces = []
    lane_w = _TB // 8
    for t in range(_TB // 128):
        s, l0 = divmod(t * 128, lane_w)
        ids_piece = ids_ref[0, s:s + 1, l0:l0 + 128]
        g = jax.lax.broadcasted_iota(jnp.int32, (_G, 128), 0)
        pieces.append((g == ids_piece).astype(tabt_ref.dtype))
    oh_t = jnp.concatenate(pieces, axis=1)
    out_ref[...] = jnp.dot(tabt_ref[...], oh_t,
                           preferred_element_type=jnp.float32)


def kernel(group_id, table):
    (B,) = group_id.shape
    num_tiles = B // _TB
    ids = group_id.astype(jnp.int32).reshape(num_tiles, 8, _TB // 8)
    tab_t = table[:_G, :_D].T

    out_t = pl.pallas_call(
        _gather_t_kernel,
        out_shape=jax.ShapeDtypeStruct((_D, B), jnp.float32),
        grid=(num_tiles,),
        in_specs=[
            pl.BlockSpec((1, 8, _TB // 8), lambda i: (i, 0, 0)),
            pl.BlockSpec((_D, _G), lambda i: (0, 0)),
        ],
        out_specs=pl.BlockSpec((_D, _TB), lambda i: (0, i)),
        compiler_params=pltpu.CompilerParams(
            dimension_semantics=("parallel",)),
    )(ids, tab_t)

    return out_t.T


# speedup vs baseline: 8.9401x; 2.1791x over previous
import jax
import jax.numpy as jnp
from jax.experimental import pallas as pl
from jax.experimental.pallas import tpu as pltpu

_G = 256
_D = 64
_TB = 8192


def _gather_t_kernel(ids_ref, tabt_ref, out_ref):
    pieces = []
    lane_w = _TB // 8
    for t in range(_TB // 128):
        s, l0 = divmod(t * 128, lane_w)
        ids_piece = ids_ref[0, s:s + 1, l0:l0 + 128]
        g = jax.lax.broadcasted_iota(jnp.int32, (_G, 128), 0)
        pieces.append((g == ids_piece).astype(tabt_ref.dtype))
    oh_t = jnp.concatenate(pieces, axis=1)
    out_ref[...] = jnp.dot(tabt_ref[...], oh_t,
                           preferred_element_type=jnp.float32)


def kernel(group_id, table):
    (B,) = group_id.shape
    num_tiles = B // _TB
    ids = group_id.astype(jnp.int32).reshape(num_tiles, 8, _TB // 8)
    tab_t = table[:_G, :_D].T

    out_t = pl.pallas_call(
        _gather_t_kernel,
        out_shape=jax.ShapeDtypeStruct((_D, B), jnp.float32),
        grid=(num_tiles,),
        in_specs=[
            pl.BlockSpec((1, 8, _TB // 8), lambda i: (i, 0, 0)),
            pl.BlockSpec((_D, _G), lambda i: (0, 0)),
        ],
        out_specs=pl.BlockSpec((_D, _TB), lambda i: (0, i)),
        compiler_params=pltpu.CompilerParams(
            dimension_semantics=("parallel",)),
    )(ids, tab_t)

    return out_t.T


# speedup vs baseline: 11.0517x; 1.2362x over previous
import jax
import jax.numpy as jnp
from jax.experimental import pallas as pl
from jax.experimental.pallas import tpu as pltpu

_G = 256
_D = 64
_TB = 16384


def _gather_t_kernel(ids_ref, tabt_ref, out_ref):
    pieces = []
    lane_w = _TB // 8
    for t in range(_TB // 128):
        s, l0 = divmod(t * 128, lane_w)
        ids_piece = ids_ref[0, s:s + 1, l0:l0 + 128]
        g = jax.lax.broadcasted_iota(jnp.int32, (_G, 128), 0)
        pieces.append((g == ids_piece).astype(tabt_ref.dtype))
    oh_t = jnp.concatenate(pieces, axis=1)
    out_ref[...] = jnp.dot(tabt_ref[...], oh_t,
                           preferred_element_type=jnp.float32)


def kernel(group_id, table):
    (B,) = group_id.shape
    num_tiles = B // _TB
    ids = group_id.astype(jnp.int32).reshape(num_tiles, 8, _TB // 8)
    tab_t = table[:_G, :_D].T

    out_t = pl.pallas_call(
        _gather_t_kernel,
        out_shape=jax.ShapeDtypeStruct((_D, B), jnp.float32),
        grid=(num_tiles,),
        in_specs=[
            pl.BlockSpec((1, 8, _TB // 8), lambda i: (i, 0, 0)),
            pl.BlockSpec((_D, _G), lambda i: (0, 0)),
        ],
        out_specs=pl.BlockSpec((_D, _TB), lambda i: (0, i)),
        compiler_params=pltpu.CompilerParams(
            dimension_semantics=("parallel",)),
    )(ids, tab_t)

    return out_t.T


# speedup vs baseline: 12.5259x; 1.1334x over previous
import jax
import jax.numpy as jnp
from jax.experimental import pallas as pl
from jax.experimental.pallas import tpu as pltpu

_G = 256
_D = 64
_TB = 32768


def _gather_t_kernel(ids_ref, tabt_ref, out_ref):
    pieces = []
    lane_w = _TB // 8
    for t in range(_TB // 128):
        s, l0 = divmod(t * 128, lane_w)
        ids_piece = ids_ref[0, s:s + 1, l0:l0 + 128]
        g = jax.lax.broadcasted_iota(jnp.int32, (_G, 128), 0)
        pieces.append((g == ids_piece).astype(tabt_ref.dtype))
    oh_t = jnp.concatenate(pieces, axis=1)
    out_ref[...] = jnp.dot(tabt_ref[...], oh_t,
                           preferred_element_type=jnp.float32)


def kernel(group_id, table):
    (B,) = group_id.shape
    num_tiles = B // _TB
    ids = group_id.astype(jnp.int32).reshape(num_tiles, 8, _TB // 8)
    tab_t = table[:_G, :_D].T

    out_t = pl.pallas_call(
        _gather_t_kernel,
        out_shape=jax.ShapeDtypeStruct((_D, B), jnp.float32),
        grid=(num_tiles,),
        in_specs=[
            pl.BlockSpec((1, 8, _TB // 8), lambda i: (i, 0, 0)),
            pl.BlockSpec((_D, _G), lambda i: (0, 0)),
        ],
        out_specs=pl.BlockSpec((_D, _TB), lambda i: (0, i)),
        compiler_params=pltpu.CompilerParams(
            dimension_semantics=("parallel",)),
    )(ids, tab_t)

    return out_t.T


# speedup vs baseline: 13.5040x; 1.0781x over previous
import jax
import jax.numpy as jnp
from jax.experimental import pallas as pl
from jax.experimental.pallas import tpu as pltpu

_G = 256
_D = 64
_TB = 65536


def _gather_t_kernel(ids_ref, tabt_ref, out_ref):
    pieces = []
    lane_w = _TB // 8
    for t in range(_TB // 128):
        s, l0 = divmod(t * 128, lane_w)
        ids_piece = ids_ref[0, s:s + 1, l0:l0 + 128]
        g = jax.lax.broadcasted_iota(jnp.int32, (_G, 128), 0)
        pieces.append((g == ids_piece).astype(tabt_ref.dtype))
    oh_t = jnp.concatenate(pieces, axis=1)
    out_ref[...] = jnp.dot(tabt_ref[...], oh_t,
                           preferred_element_type=jnp.float32)


def kernel(group_id, table):
    (B,) = group_id.shape
    num_tiles = B // _TB
    ids = group_id.astype(jnp.int32).reshape(num_tiles, 8, _TB // 8)
    tab_t = table[:_G, :_D].T

    out_t = pl.pallas_call(
        _gather_t_kernel,
        out_shape=jax.ShapeDtypeStruct((_D, B), jnp.float32),
        grid=(num_tiles,),
        in_specs=[
            pl.BlockSpec((1, 8, _TB // 8), lambda i: (i, 0, 0)),
            pl.BlockSpec((_D, _G), lambda i: (0, 0)),
        ],
        out_specs=pl.BlockSpec((_D, _TB), lambda i: (0, i)),
        compiler_params=pltpu.CompilerParams(
            dimension_semantics=("parallel",)),
    )(ids, tab_t)

    return out_t.T
